# baseline (speedup 1.0000x reference)
import functools

import jax
import jax.numpy as jnp
from jax import lax
from jax.experimental import pallas as pl
from jax.experimental.pallas import tpu as pltpu
from jax.experimental.pallas import tpu_sc as plsc

D = 128
NC = 2
NS = 16
NW = NC * NS
CHUNK = 128
N_PAD = 10240
RPT = N_PAD // NS
BLK = 1024


def _make_sc_pass(n_chunks, with_deg):
    assert n_chunks % 2 == 0
    mesh = plsc.VectorSubcoreMesh(core_axis_name="c", subcore_axis_name="s")
    out_type = [jax.ShapeDtypeStruct((NC, N_PAD, D), jnp.float32)]
    scratch = [
        pltpu.VMEM_SHARED((N_PAD, D), jnp.float32),
        pltpu.VMEM((n_chunks, CHUNK), jnp.int32),
        pltpu.VMEM((2, CHUNK), jnp.int32),
        pltpu.VMEM((2, CHUNK, D), jnp.float32),
        pltpu.SemaphoreType.DMA,
        pltpu.SemaphoreType.DMA,
        pltpu.SemaphoreType.DMA,
        pltpu.SemaphoreType.DMA,
        pltpu.SemaphoreType.DMA,
        pltpu.SemaphoreType.DMA,
    ]
    if with_deg:
        out_type.append(jax.ShapeDtypeStruct((NC, N_PAD), jnp.float32))
        scratch.append(pltpu.VMEM_SHARED((N_PAD,), jnp.float32))
        scratch.append(pltpu.VMEM((CHUNK,), jnp.float32))
        scratch.append(pltpu.SemaphoreType.DMA)
        scratch.append(pltpu.SemaphoreType.DMA)

    def body(*refs):
        e0 = e1 = None
        if with_deg:
            (table, srci, dsti, z2, z1, ones_h,
             acc_out, deg_out, acc_sh, src_v, dst_v, rows_v,
             g0, g1, d0, d1, s0, s1, deg_sh, ones_v, e0, e1) = refs
        else:
            (table, srci, dsti, z2,
             acc_out, acc_sh, src_v, dst_v, rows_v,
             g0, g1, d0, d1, s0, s1) = refs
        c = lax.axis_index("c")
        s = lax.axis_index("s")
        wid = s * NC + c
        gsems = (g0, g1)
        dsems = (d0, d1)
        ssems = (s0, s1)
        esems = (e0, e1)

        pltpu.sync_copy(srci.at[wid], src_v)
        pltpu.sync_copy(z2, acc_sh.at[pl.ds(s * RPT, RPT), :])
        if with_deg:
            pltpu.sync_copy(z1, deg_sh.at[pl.ds(s * RPT, RPT)])
            pltpu.sync_copy(ones_h, ones_v)
        plsc.subcore_barrier()

        def fetch(j, b):
            pltpu.async_copy(table.at[src_v.at[j]], rows_v.at[b], gsems[b])
            pltpu.async_copy(dsti.at[wid, j], dst_v.at[b], dsems[b])

        def wait_fetch(b):
            pltpu.make_async_copy(table.at[pl.ds(0, CHUNK), :],
                                  rows_v.at[b], gsems[b]).wait()
            pltpu.make_async_copy(dsti.at[0, 0], dst_v.at[b], dsems[b]).wait()

        def scatter(b):
            cps = [pltpu.async_copy(rows_v.at[b], acc_sh.at[dst_v.at[b]],
                                    ssems[b], add=True)]
            if with_deg:
                cps.append(pltpu.async_copy(ones_v, deg_sh.at[dst_v.at[b]],
                                            esems[b], add=True))
            return cps

        fetch(0, 0)
        fetch(1, 1)

        def step(k, carry):
            j0 = 2 * k
            wait_fetch(0)
            cp0 = scatter(0)
            wait_fetch(1)
            cp1 = scatter(1)
            for cp in cp0:
                cp.wait()

            @pl.when(k < n_chunks // 2 - 1)
            def _():
                fetch(j0 + 2, 0)

            for cp in cp1:
                cp.wait()

            @pl.when(k < n_chunks // 2 - 1)
            def _():
                fetch(j0 + 3, 1)

            return carry

        lax.fori_loop(0, n_chunks // 2, step, 0)
        plsc.subcore_barrier()

        pltpu.sync_copy(acc_sh.at[pl.ds(s * RPT, RPT), :],
                        acc_out.at[c, pl.ds(s * RPT, RPT), :])
        if with_deg:
            pltpu.sync_copy(deg_sh.at[pl.ds(s * RPT, RPT)],
                            deg_out.at[c, pl.ds(s * RPT, RPT)])

    return pl.kernel(body, out_type=tuple(out_type), mesh=mesh,
                     scratch_types=tuple(scratch))


def _combine_body(acc_ref, deg_ref, out_ref):
    i = pl.program_id(0)
    a = acc_ref[0] + acc_ref[1]
    d = deg_ref[0, pl.ds(i * BLK, BLK)] + deg_ref[1, pl.ds(i * BLK, BLK)]
    d = jnp.clip(d, 1.0, None)
    out_ref[...] = a / d[:, None]


def _combine(acc, deg):
    return pl.pallas_call(
        _combine_body,
        grid=(N_PAD // BLK,),
        in_specs=[
            pl.BlockSpec((NC, BLK, D), lambda i: (0, i, 0)),
            pl.BlockSpec((NC, N_PAD), lambda i: (0, 0)),
        ],
        out_specs=pl.BlockSpec((BLK, D), lambda i: (i, 0)),
        out_shape=jax.ShapeDtypeStruct((N_PAD, D), jnp.float32),
    )(acc, deg)


def _final_body(x_ref, h1_ref, acc2_ref, deg_ref, wt_ref, b_ref, out_ref):
    i = pl.program_id(0)
    d = deg_ref[0, pl.ds(i * BLK, BLK)] + deg_ref[1, pl.ds(i * BLK, BLK)]
    d = jnp.clip(d, 1.0, None)
    h2 = (acc2_ref[0] + acc2_ref[1]) / d[:, None]
    r = jnp.dot(x_ref[...], wt_ref[pl.ds(0, D), :],
                preferred_element_type=jnp.float32, precision="highest")
    r += jnp.dot(h1_ref[...], wt_ref[pl.ds(D, D), :],
                 preferred_element_type=jnp.float32, precision="highest")
    r += jnp.dot(h2, wt_ref[pl.ds(2 * D, D), :],
                 preferred_element_type=jnp.float32, precision="highest")
    out_ref[...] = r + b_ref[...]


def _final(x_pad, hop1, acc2, deg, wt, b2):
    return pl.pallas_call(
        _final_body,
        grid=(N_PAD // BLK,),
        in_specs=[
            pl.BlockSpec((BLK, D), lambda i: (i, 0)),
            pl.BlockSpec((BLK, D), lambda i: (i, 0)),
            pl.BlockSpec((NC, BLK, D), lambda i: (0, i, 0)),
            pl.BlockSpec((NC, N_PAD), lambda i: (0, 0)),
            pl.BlockSpec((3 * D, D), lambda i: (0, 0)),
            pl.BlockSpec((1, D), lambda i: (0, 0)),
        ],
        out_specs=pl.BlockSpec((BLK, D), lambda i: (i, 0)),
        out_shape=jax.ShapeDtypeStruct((N_PAD, D), jnp.float32),
    )(x_pad, hop1, acc2, deg, wt, b2)


def kernel(x, edge_index, W, b):
    n = x.shape[0]
    e = edge_index.shape[1]
    n_chunks = -(-e // (NW * CHUNK))
    n_chunks += n_chunks % 2
    e_pad = NW * n_chunks * CHUNK

    ei = edge_index.astype(jnp.int32)
    src = jnp.concatenate([ei[0], jnp.zeros((e_pad - e,), jnp.int32)])
    dst = jnp.concatenate([ei[1], jnp.full((e_pad - e,), N_PAD - 1, jnp.int32)])
    src3 = src.reshape(NW, n_chunks, CHUNK)
    dst3 = dst.reshape(NW, n_chunks, CHUNK)
    x_pad = jnp.concatenate([x, jnp.zeros((N_PAD - n, D), x.dtype)])
    z2 = jnp.zeros((RPT, D), jnp.float32)
    z1 = jnp.zeros((RPT,), jnp.float32)
    ones_h = jnp.ones((CHUNK,), jnp.float32)
    wt = W.T.astype(jnp.float32)
    b2 = b.reshape(1, D).astype(jnp.float32)

    pass1 = _make_sc_pass(n_chunks, with_deg=True)
    pass2 = _make_sc_pass(n_chunks, with_deg=False)

    acc1, deg = pass1(x_pad, src3, dst3, z2, z1, ones_h)
    hop1 = _combine(acc1, deg)
    (acc2,) = pass2(hop1, src3, dst3, z2)
    out_pad = _final(x_pad, hop1, acc2, deg, wt, b2)
    return out_pad[:n]


# speedup vs baseline: 1.2528x; 1.2528x over previous
import functools

import jax
import jax.numpy as jnp
from jax import lax
from jax.experimental import pallas as pl
from jax.experimental.pallas import tpu as pltpu
from jax.experimental.pallas import tpu_sc as plsc

D = 128
NC = 2
NS = 16
CHUNK = 128
N_PAD = 10240
RPT = N_PAD // NS
BLK = 1024
K0_FRAC = 0.80


def _split(e):
    per_pair = -(-(-(-e // CHUNK)) // NS)
    k0 = max(1, min(per_pair, int(round(per_pair * K0_FRAC))))
    k1 = per_pair - k0
    return k0, k1


def _make_sc_pass(k0, k1, with_deg):
    mesh = plsc.VectorSubcoreMesh(core_axis_name="c", subcore_axis_name="s")
    out_type = [jax.ShapeDtypeStruct((NC, N_PAD, D), jnp.float32)]
    scratch = [
        pltpu.VMEM_SHARED((N_PAD, D), jnp.float32),
        pltpu.VMEM((CHUNK,), jnp.int32),
        pltpu.VMEM((CHUNK,), jnp.int32),
        pltpu.VMEM((CHUNK, D), jnp.float32),
        pltpu.SemaphoreType.DMA,
    ]
    if with_deg:
        out_type.append(jax.ShapeDtypeStruct((NC, N_PAD), jnp.float32))
        scratch.append(pltpu.VMEM_SHARED((N_PAD,), jnp.float32))
        scratch.append(pltpu.VMEM((CHUNK,), jnp.float32))

    def body(*refs):
        if with_deg:
            (table, srci, dsti, z2, z1, ones_h,
             acc_out, deg_out, acc_sh, src_v, dst_v, rows_v, sem,
             deg_sh, ones_v) = refs
        else:
            (table, srci, dsti, z2,
             acc_out, acc_sh, src_v, dst_v, rows_v, sem) = refs
        c = lax.axis_index("c")
        s = lax.axis_index("s")

        pltpu.sync_copy(z2, acc_sh.at[pl.ds(s * RPT, RPT), :])
        if with_deg:
            pltpu.sync_copy(z1, deg_sh.at[pl.ds(s * RPT, RPT)])
            pltpu.sync_copy(ones_h, ones_v)
        plsc.subcore_barrier()

        def run(base, k):
            def step(j, carry):
                pltpu.sync_copy(srci.at[base + j], src_v)
                pltpu.sync_copy(dsti.at[base + j], dst_v)
                pltpu.async_copy(table.at[src_v], rows_v, sem).wait()
                pltpu.sync_copy(rows_v, acc_sh.at[dst_v], add=True)
                if with_deg:
                    pltpu.sync_copy(ones_v, deg_sh.at[dst_v], add=True)
                return carry

            lax.fori_loop(0, k, step, 0)

        if k0 > 0:
            @pl.when(c == 0)
            def _():
                run(s * k0, k0)
        if k1 > 0:
            @pl.when(c == 1)
            def _():
                run(NS * k0 + s * k1, k1)

        plsc.subcore_barrier()

        pltpu.sync_copy(acc_sh.at[pl.ds(s * RPT, RPT), :],
                        acc_out.at[c, pl.ds(s * RPT, RPT), :])
        if with_deg:
            pltpu.sync_copy(deg_sh.at[pl.ds(s * RPT, RPT)],
                            deg_out.at[c, pl.ds(s * RPT, RPT)])

    return pl.kernel(body, out_type=tuple(out_type), mesh=mesh,
                     scratch_types=tuple(scratch))


def _combine_body(acc_ref, deg_ref, out_ref):
    i = pl.program_id(0)
    a = acc_ref[0] + acc_ref[1]
    d = deg_ref[0, pl.ds(i * BLK, BLK)] + deg_ref[1, pl.ds(i * BLK, BLK)]
    d = jnp.clip(d, 1.0, None)
    out_ref[...] = a / d[:, None]


def _combine(acc, deg):
    return pl.pallas_call(
        _combine_body,
        grid=(N_PAD // BLK,),
        in_specs=[
            pl.BlockSpec((NC, BLK, D), lambda i: (0, i, 0)),
            pl.BlockSpec((NC, N_PAD), lambda i: (0, 0)),
        ],
        out_specs=pl.BlockSpec((BLK, D), lambda i: (i, 0)),
        out_shape=jax.ShapeDtypeStruct((N_PAD, D), jnp.float32),
    )(acc, deg)


def _final_body(x_ref, h1_ref, acc2_ref, deg_ref, wt_ref, b_ref, out_ref):
    i = pl.program_id(0)
    d = deg_ref[0, pl.ds(i * BLK, BLK)] + deg_ref[1, pl.ds(i * BLK, BLK)]
    d = jnp.clip(d, 1.0, None)
    h2 = (acc2_ref[0] + acc2_ref[1]) / d[:, None]
    r = jnp.dot(x_ref[...], wt_ref[pl.ds(0, D), :],
                preferred_element_type=jnp.float32, precision="highest")
    r += jnp.dot(h1_ref[...], wt_ref[pl.ds(D, D), :],
                 preferred_element_type=jnp.float32, precision="highest")
    r += jnp.dot(h2, wt_ref[pl.ds(2 * D, D), :],
                 preferred_element_type=jnp.float32, precision="highest")
    out_ref[...] = r + b_ref[...]


def _final(x_pad, hop1, acc2, deg, wt, b2):
    return pl.pallas_call(
        _final_body,
        grid=(N_PAD // BLK,),
        in_specs=[
            pl.BlockSpec((BLK, D), lambda i: (i, 0)),
            pl.BlockSpec((BLK, D), lambda i: (i, 0)),
            pl.BlockSpec((NC, BLK, D), lambda i: (0, i, 0)),
            pl.BlockSpec((NC, N_PAD), lambda i: (0, 0)),
            pl.BlockSpec((3 * D, D), lambda i: (0, 0)),
            pl.BlockSpec((1, D), lambda i: (0, 0)),
        ],
        out_specs=pl.BlockSpec((BLK, D), lambda i: (i, 0)),
        out_shape=jax.ShapeDtypeStruct((N_PAD, D), jnp.float32),
    )(x_pad, hop1, acc2, deg, wt, b2)


def kernel(x, edge_index, W, b):
    n = x.shape[0]
    e = edge_index.shape[1]
    k0, k1 = _split(e)
    tot = NS * (k0 + k1)
    e_pad = tot * CHUNK

    ei = edge_index.astype(jnp.int32)
    src = jnp.concatenate([ei[0], jnp.zeros((e_pad - e,), jnp.int32)])
    dst = jnp.concatenate([ei[1], jnp.full((e_pad - e,), N_PAD - 1, jnp.int32)])
    src2 = src.reshape(tot, CHUNK)
    dst2 = dst.reshape(tot, CHUNK)
    x_pad = jnp.concatenate([x, jnp.zeros((N_PAD - n, D), x.dtype)])
    z2 = jnp.zeros((RPT, D), jnp.float32)
    z1 = jnp.zeros((RPT,), jnp.float32)
    ones_h = jnp.ones((CHUNK,), jnp.float32)
    wt = W.T.astype(jnp.float32)
    b2 = b.reshape(1, D).astype(jnp.float32)

    pass1 = _make_sc_pass(k0, k1, with_deg=True)
    pass2 = _make_sc_pass(k0, k1, with_deg=False)

    acc1, deg = pass1(x_pad, src2, dst2, z2, z1, ones_h)
    hop1 = _combine(acc1, deg)
    (acc2,) = pass2(hop1, src2, dst2, z2)
    out_pad = _final(x_pad, hop1, acc2, deg, wt, b2)
    return out_pad[:n]


# speedup vs baseline: 1.8905x; 1.5090x over previous
import functools

import jax
import jax.numpy as jnp
from jax import lax
from jax.experimental import pallas as pl
from jax.experimental.pallas import tpu as pltpu
from jax.experimental.pallas import tpu_sc as plsc

D = 128
NC = 2
NS = 16
CHUNK = 128
N_PAD = 10240
RPT = N_PAD // NS
BLK = 1024
K0_FRAC = 0.77


def _split(e):
    per_pair = -(-(-(-e // CHUNK)) // NS)
    per_pair += per_pair % 2
    k0 = 2 * int(round(per_pair * K0_FRAC / 2))
    k0 = max(2, min(per_pair, k0))
    k1 = per_pair - k0
    return k0, k1


def _make_sc_pass(k0, k1, with_deg):
    mesh = plsc.VectorSubcoreMesh(core_axis_name="c", subcore_axis_name="s")
    out_type = [jax.ShapeDtypeStruct((NC, N_PAD, D), jnp.float32)]
    scratch = [
        pltpu.VMEM_SHARED((N_PAD, D), jnp.float32),
        pltpu.VMEM((2, 2, CHUNK), jnp.int32),
        pltpu.VMEM((2, CHUNK, D), jnp.float32),
        pltpu.SemaphoreType.DMA,
        pltpu.SemaphoreType.DMA,
        pltpu.SemaphoreType.DMA,
        pltpu.SemaphoreType.DMA,
    ]
    if with_deg:
        out_type.append(jax.ShapeDtypeStruct((NC, N_PAD), jnp.float32))
        scratch.append(pltpu.VMEM_SHARED((N_PAD,), jnp.float32))
        scratch.append(pltpu.VMEM((CHUNK,), jnp.float32))

    def body(*refs):
        if with_deg:
            (table, pki, z2, z1, ones_h,
             acc_out, deg_out, acc_sh, idx_v, rows_v, i0, i1, g0, g1,
             deg_sh, ones_v) = refs
        else:
            (table, pki, z2,
             acc_out, acc_sh, idx_v, rows_v, i0, i1, g0, g1) = refs
        c = lax.axis_index("c")
        s = lax.axis_index("s")
        isems = (i0, i1)
        gsems = (g0, g1)

        pltpu.sync_copy(z2, acc_sh.at[pl.ds(s * RPT, RPT), :])
        if with_deg:
            pltpu.sync_copy(z1, deg_sh.at[pl.ds(s * RPT, RPT)])
            pltpu.sync_copy(ones_h, ones_v)
        plsc.subcore_barrier()

        def run(base, k):
            def fetch_idx(j, b):
                pltpu.async_copy(pki.at[base + j], idx_v.at[b], isems[b])

            def wait_idx(b):
                pltpu.make_async_copy(pki.at[0], idx_v.at[b],
                                      isems[b]).wait()

            def gather(b):
                pltpu.async_copy(table.at[idx_v.at[b, 0]], rows_v.at[b],
                                 gsems[b])

            def wait_gather(b):
                pltpu.make_async_copy(table.at[pl.ds(0, CHUNK), :],
                                      rows_v.at[b], gsems[b]).wait()

            def scatter(b):
                pltpu.sync_copy(rows_v.at[b], acc_sh.at[idx_v.at[b, 1]],
                                add=True)
                if with_deg:
                    pltpu.sync_copy(ones_v, deg_sh.at[idx_v.at[b, 1]],
                                    add=True)

            fetch_idx(0, 0)
            fetch_idx(1, 1)

            def step(kk, carry):
                j0 = 2 * kk
                wait_idx(0)
                gather(0)
                wait_idx(1)
                gather(1)
                wait_gather(0)
                scatter(0)

                @pl.when(kk < k // 2 - 1)
                def _():
                    fetch_idx(j0 + 2, 0)

                wait_gather(1)
                scatter(1)

                @pl.when(kk < k // 2 - 1)
                def _():
                    fetch_idx(j0 + 3, 1)

                return carry

            lax.fori_loop(0, k // 2, step, 0)

        if k0 > 0:
            @pl.when(c == 0)
            def _():
                run(s * k0, k0)
        if k1 > 0:
            @pl.when(c == 1)
            def _():
                run(NS * k0 + s * k1, k1)

        plsc.subcore_barrier()

        pltpu.sync_copy(acc_sh.at[pl.ds(s * RPT, RPT), :],
                        acc_out.at[c, pl.ds(s * RPT, RPT), :])
        if with_deg:
            pltpu.sync_copy(deg_sh.at[pl.ds(s * RPT, RPT)],
                            deg_out.at[c, pl.ds(s * RPT, RPT)])

    return pl.kernel(body, out_type=tuple(out_type), mesh=mesh,
                     scratch_types=tuple(scratch))


def _combine_body(acc_ref, deg_ref, out_ref):
    i = pl.program_id(0)
    a = acc_ref[0] + acc_ref[1]
    d = deg_ref[0, pl.ds(i * BLK, BLK)] + deg_ref[1, pl.ds(i * BLK, BLK)]
    d = jnp.clip(d, 1.0, None)
    out_ref[...] = a / d[:, None]


def _combine(acc, deg):
    return pl.pallas_call(
        _combine_body,
        grid=(N_PAD // BLK,),
        in_specs=[
            pl.BlockSpec((NC, BLK, D), lambda i: (0, i, 0)),
            pl.BlockSpec((NC, N_PAD), lambda i: (0, 0)),
        ],
        out_specs=pl.BlockSpec((BLK, D), lambda i: (i, 0)),
        out_shape=jax.ShapeDtypeStruct((N_PAD, D), jnp.float32),
    )(acc, deg)


def _final_body(x_ref, h1_ref, acc2_ref, deg_ref, wt_ref, b_ref, out_ref):
    i = pl.program_id(0)
    d = deg_ref[0, pl.ds(i * BLK, BLK)] + deg_ref[1, pl.ds(i * BLK, BLK)]
    d = jnp.clip(d, 1.0, None)
    h2 = (acc2_ref[0] + acc2_ref[1]) / d[:, None]
    r = jnp.dot(x_ref[...], wt_ref[pl.ds(0, D), :],
                preferred_element_type=jnp.float32, precision="highest")
    r += jnp.dot(h1_ref[...], wt_ref[pl.ds(D, D), :],
                 preferred_element_type=jnp.float32, precision="highest")
    r += jnp.dot(h2, wt_ref[pl.ds(2 * D, D), :],
                 preferred_element_type=jnp.float32, precision="highest")
    out_ref[...] = r + b_ref[...]


def _final(x_pad, hop1, acc2, deg, wt, b2):
    return pl.pallas_call(
        _final_body,
        grid=(N_PAD // BLK,),
        in_specs=[
            pl.BlockSpec((BLK, D), lambda i: (i, 0)),
            pl.BlockSpec((BLK, D), lambda i: (i, 0)),
            pl.BlockSpec((NC, BLK, D), lambda i: (0, i, 0)),
            pl.BlockSpec((NC, N_PAD), lambda i: (0, 0)),
            pl.BlockSpec((3 * D, D), lambda i: (0, 0)),
            pl.BlockSpec((1, D), lambda i: (0, 0)),
        ],
        out_specs=pl.BlockSpec((BLK, D), lambda i: (i, 0)),
        out_shape=jax.ShapeDtypeStruct((N_PAD, D), jnp.float32),
    )(x_pad, hop1, acc2, deg, wt, b2)


def kernel(x, edge_index, W, b):
    n = x.shape[0]
    e = edge_index.shape[1]
    k0, k1 = _split(e)
    tot = NS * (k0 + k1)
    e_pad = tot * CHUNK

    ei = edge_index.astype(jnp.int32)
    src = jnp.concatenate([ei[0], jnp.zeros((e_pad - e,), jnp.int32)])
    dst = jnp.concatenate([ei[1], jnp.full((e_pad - e,), N_PAD - 1, jnp.int32)])
    pk = jnp.stack([src.reshape(tot, CHUNK), dst.reshape(tot, CHUNK)], axis=1)
    x_pad = jnp.concatenate([x, jnp.zeros((N_PAD - n, D), x.dtype)])
    z2 = jnp.zeros((RPT, D), jnp.float32)
    z1 = jnp.zeros((RPT,), jnp.float32)
    ones_h = jnp.ones((CHUNK,), jnp.float32)
    wt = W.T.astype(jnp.float32)
    b2 = b.reshape(1, D).astype(jnp.float32)

    pass1 = _make_sc_pass(k0, k1, with_deg=True)
    pass2 = _make_sc_pass(k0, k1, with_deg=False)

    acc1, deg = pass1(x_pad, pk, z2, z1, ones_h)
    hop1 = _combine(acc1, deg)
    (acc2,) = pass2(hop1, pk, z2)
    out_pad = _final(x_pad, hop1, acc2, deg, wt, b2)
    return out_pad[:n]


# speedup vs baseline: 1.9094x; 1.0100x over previous
import functools

import jax
import jax.numpy as jnp
from jax import lax
from jax.experimental import pallas as pl
from jax.experimental.pallas import tpu as pltpu
from jax.experimental.pallas import tpu_sc as plsc

D = 128
NC = 2
NS = 16
CHUNK = 128
N_PAD = 10240
RPT = N_PAD // NS
BLK = 1024
K0_FRAC = 0.785


def _split(e):
    per_pair = -(-(-(-e // CHUNK)) // NS)
    per_pair += per_pair % 2
    k0 = 2 * int(round(per_pair * K0_FRAC / 2))
    k0 = max(2, min(per_pair, k0))
    k1 = per_pair - k0
    return k0, k1


def _make_sc_pass(k0, k1, with_deg):
    mesh = plsc.VectorSubcoreMesh(core_axis_name="c", subcore_axis_name="s")
    out_type = [jax.ShapeDtypeStruct((NC, N_PAD, D), jnp.float32)]
    scratch = [
        pltpu.VMEM_SHARED((N_PAD, D), jnp.float32),
        pltpu.VMEM((2, 2, CHUNK), jnp.int32),
        pltpu.VMEM((2, CHUNK, D), jnp.float32),
        pltpu.SemaphoreType.DMA,
        pltpu.SemaphoreType.DMA,
        pltpu.SemaphoreType.DMA,
        pltpu.SemaphoreType.DMA,
    ]
    if with_deg:
        out_type.append(jax.ShapeDtypeStruct((NC, N_PAD), jnp.float32))
        scratch.append(pltpu.VMEM_SHARED((N_PAD,), jnp.float32))
        scratch.append(pltpu.VMEM((CHUNK,), jnp.float32))

    def body(*refs):
        if with_deg:
            (table, pki, z2, z1, ones_h,
             acc_out, deg_out, acc_sh, idx_v, rows_v, i0, i1, g0, g1,
             deg_sh, ones_v) = refs
        else:
            (table, pki, z2,
             acc_out, acc_sh, idx_v, rows_v, i0, i1, g0, g1) = refs
        c = lax.axis_index("c")
        s = lax.axis_index("s")
        isems = (i0, i1)
        gsems = (g0, g1)

        pltpu.sync_copy(z2, acc_sh.at[pl.ds(s * RPT, RPT), :])
        if with_deg:
            pltpu.sync_copy(z1, deg_sh.at[pl.ds(s * RPT, RPT)])
            pltpu.sync_copy(ones_h, ones_v)
        plsc.subcore_barrier()

        def run(base, k):
            def fetch_idx(j, b):
                pltpu.async_copy(pki.at[base + j], idx_v.at[b], isems[b])

            def wait_idx(b):
                pltpu.make_async_copy(pki.at[0], idx_v.at[b],
                                      isems[b]).wait()

            def gather(b):
                pltpu.async_copy(table.at[idx_v.at[b, 0]], rows_v.at[b],
                                 gsems[b])

            def wait_gather(b):
                pltpu.make_async_copy(table.at[pl.ds(0, CHUNK), :],
                                      rows_v.at[b], gsems[b]).wait()

            def scatter(b):
                pltpu.sync_copy(rows_v.at[b], acc_sh.at[idx_v.at[b, 1]],
                                add=True)
                if with_deg:
                    pltpu.sync_copy(ones_v, deg_sh.at[idx_v.at[b, 1]],
                                    add=True)

            fetch_idx(0, 0)
            fetch_idx(1, 1)

            def step(kk, carry):
                j0 = 2 * kk
                wait_idx(0)
                gather(0)
                wait_idx(1)
                gather(1)
                wait_gather(0)
                scatter(0)

                @pl.when(kk < k // 2 - 1)
                def _():
                    fetch_idx(j0 + 2, 0)

                wait_gather(1)
                scatter(1)

                @pl.when(kk < k // 2 - 1)
                def _():
                    fetch_idx(j0 + 3, 1)

                return carry

            lax.fori_loop(0, k // 2, step, 0)

        if k0 > 0:
            @pl.when(c == 0)
            def _():
                run(s * k0, k0)
        if k1 > 0:
            @pl.when(c == 1)
            def _():
                run(NS * k0 + s * k1, k1)

        plsc.subcore_barrier()

        pltpu.sync_copy(acc_sh.at[pl.ds(s * RPT, RPT), :],
                        acc_out.at[c, pl.ds(s * RPT, RPT), :])
        if with_deg:
            pltpu.sync_copy(deg_sh.at[pl.ds(s * RPT, RPT)],
                            deg_out.at[c, pl.ds(s * RPT, RPT)])

    return pl.kernel(body, out_type=tuple(out_type), mesh=mesh,
                     scratch_types=tuple(scratch))


def _combine_body(acc_ref, deg_ref, out_ref):
    i = pl.program_id(0)
    a = acc_ref[0] + acc_ref[1]
    d = deg_ref[0, pl.ds(i * BLK, BLK)] + deg_ref[1, pl.ds(i * BLK, BLK)]
    d = jnp.clip(d, 1.0, None)
    out_ref[...] = a / d[:, None]


def _combine(acc, deg):
    return pl.pallas_call(
        _combine_body,
        grid=(N_PAD // BLK,),
        in_specs=[
            pl.BlockSpec((NC, BLK, D), lambda i: (0, i, 0)),
            pl.BlockSpec((NC, N_PAD), lambda i: (0, 0)),
        ],
        out_specs=pl.BlockSpec((BLK, D), lambda i: (i, 0)),
        out_shape=jax.ShapeDtypeStruct((N_PAD, D), jnp.float32),
    )(acc, deg)


def _final_body(x_ref, h1_ref, acc2_ref, deg_ref, wt_ref, b_ref, out_ref):
    i = pl.program_id(0)
    d = deg_ref[0, pl.ds(i * BLK, BLK)] + deg_ref[1, pl.ds(i * BLK, BLK)]
    d = jnp.clip(d, 1.0, None)
    h2 = (acc2_ref[0] + acc2_ref[1]) / d[:, None]
    r = jnp.dot(x_ref[...], wt_ref[pl.ds(0, D), :],
                preferred_element_type=jnp.float32, precision="highest")
    r += jnp.dot(h1_ref[...], wt_ref[pl.ds(D, D), :],
                 preferred_element_type=jnp.float32, precision="highest")
    r += jnp.dot(h2, wt_ref[pl.ds(2 * D, D), :],
                 preferred_element_type=jnp.float32, precision="highest")
    out_ref[...] = r + b_ref[...]


def _final(x_pad, hop1, acc2, deg, wt, b2):
    return pl.pallas_call(
        _final_body,
        grid=(N_PAD // BLK,),
        in_specs=[
            pl.BlockSpec((BLK, D), lambda i: (i, 0)),
            pl.BlockSpec((BLK, D), lambda i: (i, 0)),
            pl.BlockSpec((NC, BLK, D), lambda i: (0, i, 0)),
            pl.BlockSpec((NC, N_PAD), lambda i: (0, 0)),
            pl.BlockSpec((3 * D, D), lambda i: (0, 0)),
            pl.BlockSpec((1, D), lambda i: (0, 0)),
        ],
        out_specs=pl.BlockSpec((BLK, D), lambda i: (i, 0)),
        out_shape=jax.ShapeDtypeStruct((N_PAD, D), jnp.float32),
    )(x_pad, hop1, acc2, deg, wt, b2)


def kernel(x, edge_index, W, b):
    n = x.shape[0]
    e = edge_index.shape[1]
    k0, k1 = _split(e)
    tot = NS * (k0 + k1)
    e_pad = tot * CHUNK

    ei = edge_index.astype(jnp.int32)
    src = jnp.concatenate([ei[0], jnp.zeros((e_pad - e,), jnp.int32)])
    dst = jnp.concatenate([ei[1], jnp.full((e_pad - e,), N_PAD - 1, jnp.int32)])
    pk = jnp.stack([src.reshape(tot, CHUNK), dst.reshape(tot, CHUNK)], axis=1)
    x_pad = jnp.concatenate([x, jnp.zeros((N_PAD - n, D), x.dtype)])
    z2 = jnp.zeros((RPT, D), jnp.float32)
    z1 = jnp.zeros((RPT,), jnp.float32)
    ones_h = jnp.ones((CHUNK,), jnp.float32)
    wt = W.T.astype(jnp.float32)
    b2 = b.reshape(1, D).astype(jnp.float32)

    pass1 = _make_sc_pass(k0, k1, with_deg=True)
    pass2 = _make_sc_pass(k0, k1, with_deg=False)

    acc1, deg = pass1(x_pad, pk, z2, z1, ones_h)
    hop1 = _combine(acc1, deg)
    (acc2,) = pass2(hop1, pk, z2)
    out_pad = _final(x_pad, hop1, acc2, deg, wt, b2)
    return out_pad[:n]
